# baseline (speedup 1.0000x reference)
import functools

import jax
import jax.numpy as jnp
from jax import lax
from jax.experimental import pallas as pl
from jax.experimental.pallas import tpu as pltpu
from jax.experimental.pallas import tpu_sc as plsc

N_NODES = 50000
N_EDGES = 800000
D_IN = 128
D_H = 64

NUM_CORES = 2
NUM_TILES = 16
NCHUNKS = 4
CHUNK = N_NODES // NCHUNKS
CHUNK_PAD = 12544
ROWS_PER_TILE = CHUNK_PAD // NUM_TILES
ZROWS = 112
LAST_TILE_ROWS = CHUNK - (NUM_TILES - 1) * ROWS_PER_TILE

EC = 128
BLK_ROWS = 14
EB = EC * BLK_ROWS
E_PAD = 802816
E2D_ROWS = E_PAD // EC
STRIPE_ROWS = E2D_ROWS // NUM_TILES
NBLK = STRIPE_ROWS // BLK_ROWS
CAP = 2048
NCH = CAP // EC
SLAG = 3
NSLOT = 7

RB = 400


def _make_segment_sum():
    mesh = plsc.VectorSubcoreMesh(core_axis_name="c", subcore_axis_name="s")

    @functools.partial(
        pl.kernel,
        mesh=mesh,
        out_type=jax.ShapeDtypeStruct((N_NODES // 2, 2 * D_H), jnp.float32),
        scratch_types=[
            pltpu.VMEM((2 * BLK_ROWS, EC), jnp.int32),
            pltpu.VMEM((2 * BLK_ROWS, EC), jnp.int32),
            pltpu.VMEM((NCH, EC), jnp.int32),
            pltpu.VMEM((NCH, EC), jnp.int32),
            pltpu.VMEM((NSLOT * EC, D_H), jnp.float32),
            pltpu.VMEM((ZROWS, D_H), jnp.float32),
            pltpu.VMEM_SHARED((CHUNK_PAD, D_H), jnp.float32),
            pltpu.SemaphoreType.DMA,
            pltpu.SemaphoreType.DMA,
            pltpu.SemaphoreType.DMA,
        ],
        compiler_params=pltpu.CompilerParams(
            use_tc_tiling_on_sc=False, needs_layout_passes=False),
    )
    def seg_sum(m_hbm, src_hbm, dst_hbm, out_hbm, sbuf, dbuf, csrc,
                cdst, rows, zbuf, acc, sem_g, sem_s, sem_e):
        c = lax.axis_index("c")
        s = lax.axis_index("s")
        zero16 = jnp.zeros((16,), jnp.float32)
        for i in range(ZROWS):
            for col in range(D_H // 16):
                zbuf[i, pl.ds(col * 16, 16)] = zero16

        def slotref(q):
            sl = lax.rem(q, NSLOT)
            return rows.at[pl.ds(pl.multiple_of(sl * EC, EC), EC)]

        def fire_gather(q):
            pltpu.async_copy(m_hbm.at[csrc.at[q & (NCH - 1)]], slotref(q),
                             sem_g)

        def drain_gather(q):
            pltpu.make_async_copy(m_hbm.at[csrc.at[q & (NCH - 1)]],
                                  slotref(q), sem_g).wait()

        def fire_scatter(q):
            pltpu.async_copy(slotref(q), acc.at[cdst.at[q & (NCH - 1)]],
                             sem_s, add=True)

        def drain_scatter(q):
            pltpu.make_async_copy(slotref(q),
                                  acc.at[cdst.at[q & (NCH - 1)]],
                                  sem_s).wait()

        def pump(gq):
            @pl.when(gq >= 2 * SLAG)
            def _():
                drain_scatter(gq - 2 * SLAG)

            fire_gather(gq)

            @pl.when(gq >= SLAG)
            def _():
                drain_gather(gq - SLAG)
                fire_scatter(gq - SLAG)

        def chunk_body(k, _):
            base = (2 * c + k) * CHUNK
            trash = CHUNK + s

            for j in range(ROWS_PER_TILE // ZROWS):
                pltpu.sync_copy(
                    zbuf, acc.at[pl.ds(s * ROWS_PER_TILE + j * ZROWS, ZROWS)])

            def fire_edge_load(b):
                row_off = s * STRIPE_ROWS + b * BLK_ROWS
                buf = pl.ds((b & 1) * BLK_ROWS, BLK_ROWS)
                pltpu.async_copy(src_hbm.at[pl.ds(row_off, BLK_ROWS)],
                                 sbuf.at[buf], sem_e)
                pltpu.async_copy(dst_hbm.at[pl.ds(row_off, BLK_ROWS)],
                                 dbuf.at[buf], sem_e)

            def drain_edge_load(b):
                row_off = s * STRIPE_ROWS + b * BLK_ROWS
                buf = pl.ds((b & 1) * BLK_ROWS, BLK_ROWS)
                pltpu.make_async_copy(src_hbm.at[pl.ds(row_off, BLK_ROWS)],
                                      sbuf.at[buf], sem_e).wait()
                pltpu.make_async_copy(dst_hbm.at[pl.ds(row_off, BLK_ROWS)],
                                      dbuf.at[buf], sem_e).wait()

            fire_edge_load(0)
            plsc.subcore_barrier()

            def blk(b, carry):
                off0, gq0 = carry

                @pl.when(b + 1 < NBLK)
                def _():
                    fire_edge_load(b + 1)

                drain_edge_load(b)
                rbase = (b & 1) * BLK_ROWS

                def group(g, carry):
                    off, gq = carry
                    r = rbase + (g >> 3)
                    col = pl.multiple_of((g & 7) * 16, 16)
                    s16 = sbuf[r, pl.ds(col, 16)]
                    d16 = dbuf[r, pl.ds(col, 16)]
                    ok = (d16 >= base) & (d16 < base + CHUNK)
                    okc = ok.astype(jnp.int32)
                    inc = jnp.cumsum(okc)
                    pos = (off + inc - 1) & (CAP - 1)
                    prow = pos >> 7
                    pcol = pos & (EC - 1)
                    two = s16 << 1
                    g16 = jnp.where(s16 >= N_NODES // 2,
                                    two - (N_NODES - 1), two)
                    plsc.store_scatter(csrc, [prow, pcol], g16, mask=ok)
                    plsc.store_scatter(cdst, [prow, pcol], d16 - base,
                                       mask=ok)
                    off = off + jnp.sum(okc, axis=0)
                    fire = (off >> 7) > gq

                    @pl.when(fire)
                    def _():
                        pump(gq)

                    return (off, jnp.where(fire, gq + 1, gq))

                return lax.fori_loop(0, BLK_ROWS * EC // 16, group,
                                     (off0, gq0))

            off, gq = lax.fori_loop(
                0, NBLK, blk, (jnp.int32(0), jnp.int32(0)))

            target = ((off + EC - 1) // EC) * EC

            def padg(i, _):
                pos_l = off + i * 16 + lax.iota(jnp.int32, 16)
                mk = pos_l < target
                posm = pos_l & (CAP - 1)
                prow = posm >> 7
                pcol = posm & (EC - 1)
                zi = jnp.zeros((16,), jnp.int32)
                plsc.store_scatter(csrc, [prow, pcol], zi, mask=mk)
                plsc.store_scatter(cdst, [prow, pcol], zi + trash, mask=mk)
                return 0

            lax.fori_loop(0, EC // 16, padg, 0)
            nchunks = target // EC

            def fire_rest(_, gq2):
                pump(gq2)
                return gq2 + 1

            gq = lax.fori_loop(0, nchunks - gq, fire_rest, gq)

            def ep(i, g2):
                @pl.when((g2 >= 2 * SLAG) & (g2 - 2 * SLAG < nchunks))
                def _():
                    drain_scatter(g2 - 2 * SLAG)

                @pl.when((g2 >= SLAG) & (g2 - SLAG < nchunks))
                def _():
                    drain_gather(g2 - SLAG)
                    fire_scatter(g2 - SLAG)

                return g2 + 1

            lax.fori_loop(0, 2 * SLAG, ep, gq)
            plsc.subcore_barrier()

            out_base = k * CHUNK + s * ROWS_PER_TILE
            col = pl.ds(c * D_H, D_H)

            @pl.when(s < NUM_TILES - 1)
            def _():
                pltpu.sync_copy(
                    acc.at[pl.ds(s * ROWS_PER_TILE, ROWS_PER_TILE)],
                    out_hbm.at[pl.ds(out_base, ROWS_PER_TILE), col])

            @pl.when(s == NUM_TILES - 1)
            def _():
                pltpu.sync_copy(
                    acc.at[pl.ds(s * ROWS_PER_TILE, LAST_TILE_ROWS)],
                    out_hbm.at[pl.ds(out_base, LAST_TILE_ROWS), col])

            plsc.subcore_barrier()
            return 0

        lax.fori_loop(0, NCHUNKS // NUM_CORES, chunk_body, 0)

    return seg_sum


_seg_sum = _make_segment_sum()


NP = N_NODES // 2
HB = 200
NHB = NP // HB


def _blkdiag(wt):
    d0, d1 = wt.shape
    z = jnp.zeros((d0, d1), jnp.float32)
    return jnp.concatenate(
        [jnp.concatenate([wt, z], axis=1), jnp.concatenate([z, wt], axis=1)],
        axis=0)


def _fc_pre(x, wp_t, b_row, wrel_bd):
    def body(x1_ref, x2_ref, wp_ref, b_ref, wr_ref, h_ref, m_ref):
        h_l = jnp.dot(x1_ref[...], wp_ref[...],
                      preferred_element_type=jnp.float32) + b_ref[...]
        h_r = jnp.dot(x2_ref[...], wp_ref[...],
                      preferred_element_type=jnp.float32) + b_ref[...]
        h = jnp.concatenate([h_l, h_r], axis=1)
        h_ref[...] = h
        m_ref[...] = jnp.dot(h, wr_ref[...], preferred_element_type=jnp.float32)

    return pl.pallas_call(
        body,
        grid=(NHB,),
        in_specs=[
            pl.BlockSpec((HB, D_IN), lambda i: (i, 0)),
            pl.BlockSpec((HB, D_IN), lambda i: (i + NHB, 0)),
            pl.BlockSpec((D_IN, D_H), lambda i: (0, 0)),
            pl.BlockSpec((1, D_H), lambda i: (0, 0)),
            pl.BlockSpec((2 * D_H, 2 * D_H), lambda i: (0, 0)),
        ],
        out_specs=[
            pl.BlockSpec((HB, 2 * D_H), lambda i: (i, 0)),
            pl.BlockSpec((HB, 2 * D_H), lambda i: (i, 0)),
        ],
        out_shape=[
            jax.ShapeDtypeStruct((NP, 2 * D_H), jnp.float32),
            jax.ShapeDtypeStruct((NP, 2 * D_H), jnp.float32),
        ],
    )(x, x, wp_t, b_row, wrel_bd)


def _gc_mid(aggp, hp_prev, wroot_bd, b_bd, wnrel_bd):
    def body(a_ref, h_ref, wr_ref, b_ref, wn_ref, o_ref, m_ref):
        t = jnp.tanh(a_ref[...] + b_ref[...] +
                     jnp.dot(h_ref[...], wr_ref[...],
                             preferred_element_type=jnp.float32))
        o_ref[...] = t
        m_ref[...] = jnp.dot(t, wn_ref[...], preferred_element_type=jnp.float32)

    return pl.pallas_call(
        body,
        grid=(NHB,),
        in_specs=[
            pl.BlockSpec((HB, 2 * D_H), lambda i: (i, 0)),
            pl.BlockSpec((HB, 2 * D_H), lambda i: (i, 0)),
            pl.BlockSpec((2 * D_H, 2 * D_H), lambda i: (0, 0)),
            pl.BlockSpec((1, 2 * D_H), lambda i: (0, 0)),
            pl.BlockSpec((2 * D_H, 2 * D_H), lambda i: (0, 0)),
        ],
        out_specs=[
            pl.BlockSpec((HB, 2 * D_H), lambda i: (i, 0)),
            pl.BlockSpec((HB, 2 * D_H), lambda i: (i, 0)),
        ],
        out_shape=[
            jax.ShapeDtypeStruct((NP, 2 * D_H), jnp.float32),
            jax.ShapeDtypeStruct((NP, 2 * D_H), jnp.float32),
        ],
    )(aggp, hp_prev, wroot_bd, b_bd, wnrel_bd)


def _gc_last(aggp, hp_prev, wroot_t, b_row):
    def body(a_ref, h_ref, wr_ref, b_ref, o_ref):
        half = pl.program_id(0) // NHB

        def compute(sl):
            return jnp.tanh(a_ref[:, sl] + b_ref[...] +
                            jnp.dot(h_ref[:, sl], wr_ref[...],
                                    preferred_element_type=jnp.float32))

        @pl.when(half == 0)
        def _():
            o_ref[...] = compute(slice(0, D_H))

        @pl.when(half == 1)
        def _():
            o_ref[...] = compute(slice(D_H, 2 * D_H))

    return pl.pallas_call(
        body,
        grid=(2 * NHB,),
        in_specs=[
            pl.BlockSpec((HB, 2 * D_H), lambda i: (i % NHB, 0)),
            pl.BlockSpec((HB, 2 * D_H), lambda i: (i % NHB, 0)),
            pl.BlockSpec((D_H, D_H), lambda i: (0, 0)),
            pl.BlockSpec((1, D_H), lambda i: (0, 0)),
        ],
        out_specs=pl.BlockSpec((HB, D_H), lambda i: (i, 0)),
        out_shape=jax.ShapeDtypeStruct((N_NODES, D_H), jnp.float32),
    )(aggp, hp_prev, wroot_t, b_row)


def kernel(x, edge_index, W_pre, b_pre, W1_rel, W1_root, b1, W2_rel, W2_root,
           b2):
    pad = E_PAD - N_EDGES
    src2d = jnp.concatenate(
        [edge_index[0], jnp.zeros((pad,), jnp.int32)]).reshape(E2D_ROWS, EC)
    dst2d = jnp.concatenate(
        [edge_index[1], jnp.full((pad,), N_NODES, jnp.int32)]
    ).reshape(E2D_ROWS, EC)

    b1d = jnp.concatenate([b1, b1]).reshape(1, 2 * D_H)

    h0p, m1p = _fc_pre(x, W_pre.T, b_pre.reshape(1, D_H), _blkdiag(W1_rel.T))
    agg1p = _seg_sum(m1p.reshape(N_NODES, D_H), src2d, dst2d)
    h1p, m2p = _gc_mid(agg1p, h0p, _blkdiag(W1_root.T), b1d,
                       _blkdiag(W2_rel.T))
    agg2p = _seg_sum(m2p.reshape(N_NODES, D_H), src2d, dst2d)
    return _gc_last(agg2p, h1p, W2_root.T, b2.reshape(1, D_H))


# speedup vs baseline: 1.0292x; 1.0292x over previous
import functools

import jax
import jax.numpy as jnp
from jax import lax
from jax.experimental import pallas as pl
from jax.experimental.pallas import tpu as pltpu
from jax.experimental.pallas import tpu_sc as plsc

N_NODES = 50000
N_EDGES = 800000
D_IN = 128
D_H = 64

NUM_CORES = 2
NUM_TILES = 16
NCHUNKS = 4
CHUNK = N_NODES // NCHUNKS
CHUNK_PAD = 12544
ROWS_PER_TILE = CHUNK_PAD // NUM_TILES
ZROWS = 112
LAST_TILE_ROWS = CHUNK - (NUM_TILES - 1) * ROWS_PER_TILE

EC = 128
BLK_ROWS = 14
EB = EC * BLK_ROWS
E_PAD = 802816
E2D_ROWS = E_PAD // EC
STRIPE_ROWS = E2D_ROWS // NUM_TILES
NBLK = STRIPE_ROWS // BLK_ROWS
CAP = 2048
NCH = CAP // EC
SLAG = 3
NSLOT = 7

RB = 400


def _make_segment_sum():
    mesh = plsc.VectorSubcoreMesh(core_axis_name="c", subcore_axis_name="s")

    @functools.partial(
        pl.kernel,
        mesh=mesh,
        out_type=jax.ShapeDtypeStruct((N_NODES // 2, 2 * D_H), jnp.float32),
        scratch_types=[
            pltpu.VMEM((2 * BLK_ROWS, EC), jnp.int32),
            pltpu.VMEM((2 * BLK_ROWS, EC), jnp.int32),
            pltpu.VMEM((NCH, EC), jnp.int32),
            pltpu.VMEM((NCH, EC), jnp.int32),
            pltpu.VMEM((NSLOT * EC, D_H), jnp.float32),
            pltpu.VMEM((ZROWS, D_H), jnp.float32),
            pltpu.VMEM_SHARED((CHUNK_PAD, D_H), jnp.float32),
            pltpu.SemaphoreType.DMA,
            pltpu.SemaphoreType.DMA,
            pltpu.SemaphoreType.DMA,
        ],
        compiler_params=pltpu.CompilerParams(
            use_tc_tiling_on_sc=False, needs_layout_passes=False),
    )
    def seg_sum(m_hbm, ed_hbm, out_hbm, sbuf, dbuf, csrc,
                cdst, rows, zbuf, acc, sem_g, sem_s, sem_e):
        c = lax.axis_index("c")
        s = lax.axis_index("s")
        zero16 = jnp.zeros((16,), jnp.float32)
        for i in range(ZROWS):
            for col in range(D_H // 16):
                zbuf[i, pl.ds(col * 16, 16)] = zero16

        def slotref(q):
            sl = lax.rem(q, NSLOT)
            return rows.at[pl.ds(pl.multiple_of(sl * EC, EC), EC)]

        def fire_gather(q):
            pltpu.async_copy(m_hbm.at[csrc.at[q & (NCH - 1)]], slotref(q),
                             sem_g)

        def drain_gather(q):
            pltpu.make_async_copy(m_hbm.at[csrc.at[q & (NCH - 1)]],
                                  slotref(q), sem_g).wait()

        def fire_scatter(q):
            pltpu.async_copy(slotref(q), acc.at[cdst.at[q & (NCH - 1)]],
                             sem_s, add=True)

        def drain_scatter(q):
            pltpu.make_async_copy(slotref(q),
                                  acc.at[cdst.at[q & (NCH - 1)]],
                                  sem_s).wait()

        def pump(gq):
            @pl.when(gq >= 2 * SLAG)
            def _():
                drain_scatter(gq - 2 * SLAG)

            fire_gather(gq)

            @pl.when(gq >= SLAG)
            def _():
                drain_gather(gq - SLAG)
                fire_scatter(gq - SLAG)

        def chunk_body(k, _):
            base = (2 * c + k) * CHUNK
            trash = CHUNK + s

            for j in range(ROWS_PER_TILE // ZROWS):
                pltpu.sync_copy(
                    zbuf, acc.at[pl.ds(s * ROWS_PER_TILE + j * ZROWS, ZROWS)])

            def fire_edge_load(b):
                row_off = s * STRIPE_ROWS + b * BLK_ROWS
                buf = pl.ds((b & 1) * BLK_ROWS, BLK_ROWS)
                pltpu.async_copy(ed_hbm.at[pl.ds(row_off, BLK_ROWS)],
                                 sbuf.at[buf], sem_e)
                pltpu.async_copy(
                    ed_hbm.at[pl.ds(E2D_ROWS + row_off, BLK_ROWS)],
                    dbuf.at[buf], sem_e)

            def drain_edge_load(b):
                row_off = s * STRIPE_ROWS + b * BLK_ROWS
                buf = pl.ds((b & 1) * BLK_ROWS, BLK_ROWS)
                pltpu.make_async_copy(ed_hbm.at[pl.ds(row_off, BLK_ROWS)],
                                      sbuf.at[buf], sem_e).wait()
                pltpu.make_async_copy(
                    ed_hbm.at[pl.ds(E2D_ROWS + row_off, BLK_ROWS)],
                    dbuf.at[buf], sem_e).wait()

            fire_edge_load(0)
            plsc.subcore_barrier()

            def blk(b, carry):
                off0, gq0 = carry

                @pl.when(b + 1 < NBLK)
                def _():
                    fire_edge_load(b + 1)

                drain_edge_load(b)
                rbase = (b & 1) * BLK_ROWS

                def group(g, carry):
                    off, gq = carry
                    r = rbase + (g >> 3)
                    col = pl.multiple_of((g & 7) * 16, 16)
                    s16 = sbuf[r, pl.ds(col, 16)]
                    d16 = dbuf[r, pl.ds(col, 16)]
                    ok = (d16 >= base) & (d16 < base + CHUNK)
                    okc = ok.astype(jnp.int32)
                    inc = jnp.cumsum(okc)
                    pos = (off + inc - 1) & (CAP - 1)
                    prow = pos >> 7
                    pcol = pos & (EC - 1)
                    two = s16 << 1
                    g16 = jnp.minimum(
                        jnp.where(s16 >= N_NODES // 2, two - (N_NODES - 1),
                                  two),
                        N_NODES - 1)
                    plsc.store_scatter(csrc, [prow, pcol], g16, mask=ok)
                    plsc.store_scatter(cdst, [prow, pcol], d16 - base,
                                       mask=ok)
                    off = off + jnp.sum(okc, axis=0)
                    fire = (off >> 7) > gq

                    @pl.when(fire)
                    def _():
                        pump(gq)

                    return (off, jnp.where(fire, gq + 1, gq))

                return lax.fori_loop(0, BLK_ROWS * EC // 16, group,
                                     (off0, gq0))

            off, gq = lax.fori_loop(
                0, NBLK, blk, (jnp.int32(0), jnp.int32(0)))

            target = ((off + EC - 1) // EC) * EC

            def padg(i, _):
                pos_l = off + i * 16 + lax.iota(jnp.int32, 16)
                mk = pos_l < target
                posm = pos_l & (CAP - 1)
                prow = posm >> 7
                pcol = posm & (EC - 1)
                zi = jnp.zeros((16,), jnp.int32)
                plsc.store_scatter(csrc, [prow, pcol], zi, mask=mk)
                plsc.store_scatter(cdst, [prow, pcol], zi + trash, mask=mk)
                return 0

            lax.fori_loop(0, EC // 16, padg, 0)
            nchunks = target // EC

            def fire_rest(_, gq2):
                pump(gq2)
                return gq2 + 1

            gq = lax.fori_loop(0, nchunks - gq, fire_rest, gq)

            def ep(i, g2):
                @pl.when((g2 >= 2 * SLAG) & (g2 - 2 * SLAG < nchunks))
                def _():
                    drain_scatter(g2 - 2 * SLAG)

                @pl.when((g2 >= SLAG) & (g2 - SLAG < nchunks))
                def _():
                    drain_gather(g2 - SLAG)
                    fire_scatter(g2 - SLAG)

                return g2 + 1

            lax.fori_loop(0, 2 * SLAG, ep, gq)
            plsc.subcore_barrier()

            out_base = k * CHUNK + s * ROWS_PER_TILE
            col = pl.ds(c * D_H, D_H)

            @pl.when(s < NUM_TILES - 1)
            def _():
                pltpu.sync_copy(
                    acc.at[pl.ds(s * ROWS_PER_TILE, ROWS_PER_TILE)],
                    out_hbm.at[pl.ds(out_base, ROWS_PER_TILE), col])

            @pl.when(s == NUM_TILES - 1)
            def _():
                pltpu.sync_copy(
                    acc.at[pl.ds(s * ROWS_PER_TILE, LAST_TILE_ROWS)],
                    out_hbm.at[pl.ds(out_base, LAST_TILE_ROWS), col])

            plsc.subcore_barrier()
            return 0

        lax.fori_loop(0, NCHUNKS // NUM_CORES, chunk_body, 0)

    return seg_sum


_seg_sum = _make_segment_sum()


NP = N_NODES // 2
HB = 200
NHB = NP // HB


def _blkdiag(wt):
    d0, d1 = wt.shape
    z = jnp.zeros((d0, d1), jnp.float32)
    return jnp.concatenate(
        [jnp.concatenate([wt, z], axis=1), jnp.concatenate([z, wt], axis=1)],
        axis=0)


def _fc_pre(x, wp_t, b_row, wrel_bd):
    def body(x1_ref, x2_ref, wp_ref, b_ref, wr_ref, h_ref, m_ref):
        h_l = jnp.dot(x1_ref[...], wp_ref[...],
                      preferred_element_type=jnp.float32) + b_ref[...]
        h_r = jnp.dot(x2_ref[...], wp_ref[...],
                      preferred_element_type=jnp.float32) + b_ref[...]
        h = jnp.concatenate([h_l, h_r], axis=1)
        h_ref[...] = h
        m_ref[...] = jnp.dot(h, wr_ref[...], preferred_element_type=jnp.float32)

    return pl.pallas_call(
        body,
        grid=(NHB,),
        in_specs=[
            pl.BlockSpec((HB, D_IN), lambda i: (i, 0)),
            pl.BlockSpec((HB, D_IN), lambda i: (i + NHB, 0)),
            pl.BlockSpec((D_IN, D_H), lambda i: (0, 0)),
            pl.BlockSpec((1, D_H), lambda i: (0, 0)),
            pl.BlockSpec((2 * D_H, 2 * D_H), lambda i: (0, 0)),
        ],
        out_specs=[
            pl.BlockSpec((HB, 2 * D_H), lambda i: (i, 0)),
            pl.BlockSpec((HB, 2 * D_H), lambda i: (i, 0)),
        ],
        out_shape=[
            jax.ShapeDtypeStruct((NP, 2 * D_H), jnp.float32),
            jax.ShapeDtypeStruct((NP, 2 * D_H), jnp.float32),
        ],
    )(x, x, wp_t, b_row, wrel_bd)


def _gc_mid(aggp, hp_prev, wroot_bd, b_bd, wnrel_bd):
    def body(a_ref, h_ref, wr_ref, b_ref, wn_ref, o_ref, m_ref):
        t = jnp.tanh(a_ref[...] + b_ref[...] +
                     jnp.dot(h_ref[...], wr_ref[...],
                             preferred_element_type=jnp.float32))
        o_ref[...] = t
        m_ref[...] = jnp.dot(t, wn_ref[...], preferred_element_type=jnp.float32)

    return pl.pallas_call(
        body,
        grid=(NHB,),
        in_specs=[
            pl.BlockSpec((HB, 2 * D_H), lambda i: (i, 0)),
            pl.BlockSpec((HB, 2 * D_H), lambda i: (i, 0)),
            pl.BlockSpec((2 * D_H, 2 * D_H), lambda i: (0, 0)),
            pl.BlockSpec((1, 2 * D_H), lambda i: (0, 0)),
            pl.BlockSpec((2 * D_H, 2 * D_H), lambda i: (0, 0)),
        ],
        out_specs=[
            pl.BlockSpec((HB, 2 * D_H), lambda i: (i, 0)),
            pl.BlockSpec((HB, 2 * D_H), lambda i: (i, 0)),
        ],
        out_shape=[
            jax.ShapeDtypeStruct((NP, 2 * D_H), jnp.float32),
            jax.ShapeDtypeStruct((NP, 2 * D_H), jnp.float32),
        ],
    )(aggp, hp_prev, wroot_bd, b_bd, wnrel_bd)


def _gc_last(aggp, hp_prev, wroot_t, b_row):
    def body(a_ref, h_ref, wr_ref, b_ref, o_ref):
        half = pl.program_id(0) // NHB

        def compute(sl):
            return jnp.tanh(a_ref[:, sl] + b_ref[...] +
                            jnp.dot(h_ref[:, sl], wr_ref[...],
                                    preferred_element_type=jnp.float32))

        @pl.when(half == 0)
        def _():
            o_ref[...] = compute(slice(0, D_H))

        @pl.when(half == 1)
        def _():
            o_ref[...] = compute(slice(D_H, 2 * D_H))

    return pl.pallas_call(
        body,
        grid=(2 * NHB,),
        in_specs=[
            pl.BlockSpec((HB, 2 * D_H), lambda i: (i % NHB, 0)),
            pl.BlockSpec((HB, 2 * D_H), lambda i: (i % NHB, 0)),
            pl.BlockSpec((D_H, D_H), lambda i: (0, 0)),
            pl.BlockSpec((1, D_H), lambda i: (0, 0)),
        ],
        out_specs=pl.BlockSpec((HB, D_H), lambda i: (i, 0)),
        out_shape=jax.ShapeDtypeStruct((N_NODES, D_H), jnp.float32),
    )(aggp, hp_prev, wroot_t, b_row)


def kernel(x, edge_index, W_pre, b_pre, W1_rel, W1_root, b1, W2_rel, W2_root,
           b2):
    ed2d = jnp.pad(edge_index, ((0, 0), (0, E_PAD - N_EDGES)),
                   constant_values=N_NODES).reshape(2 * E2D_ROWS, EC)

    b1d = jnp.concatenate([b1, b1]).reshape(1, 2 * D_H)

    h0p, m1p = _fc_pre(x, W_pre.T, b_pre.reshape(1, D_H), _blkdiag(W1_rel.T))
    agg1p = _seg_sum(m1p.reshape(N_NODES, D_H), ed2d)
    h1p, m2p = _gc_mid(agg1p, h0p, _blkdiag(W1_root.T), b1d,
                       _blkdiag(W2_rel.T))
    agg2p = _seg_sum(m2p.reshape(N_NODES, D_H), ed2d)
    return _gc_last(agg2p, h1p, W2_root.T, b2.reshape(1, D_H))


# speedup vs baseline: 1.0869x; 1.0561x over previous
import functools

import jax
import jax.numpy as jnp
from jax import lax
from jax.experimental import pallas as pl
from jax.experimental.pallas import tpu as pltpu
from jax.experimental.pallas import tpu_sc as plsc

N_NODES = 50000
N_EDGES = 800000
D_IN = 128
D_H = 64

NUM_CORES = 2
NUM_TILES = 16
NCHUNKS = 4
CHUNK = N_NODES // NCHUNKS
CHUNK_PAD = 12544
ROWS_PER_TILE = CHUNK_PAD // NUM_TILES
ZROWS = 112
LAST_TILE_ROWS = CHUNK - (NUM_TILES - 1) * ROWS_PER_TILE

EC = 128
BLK_ROWS = 14
EB = EC * BLK_ROWS
E_PAD = 802816
E2D_ROWS = E_PAD // EC
STRIPE_ROWS = E2D_ROWS // NUM_TILES
NBLK = STRIPE_ROWS // BLK_ROWS
CAP = 2048
NCH = CAP // EC
SLAG = 3
NSLOT = 7

RB = 400


def _make_segment_sum():
    mesh = plsc.VectorSubcoreMesh(core_axis_name="c", subcore_axis_name="s")

    @functools.partial(
        pl.kernel,
        mesh=mesh,
        out_type=jax.ShapeDtypeStruct((N_NODES, D_H), jnp.float32),
        scratch_types=[
            pltpu.VMEM((2 * BLK_ROWS, EC), jnp.int32),
            pltpu.VMEM((2 * BLK_ROWS, EC), jnp.int32),
            pltpu.VMEM((NCH, EC), jnp.int32),
            pltpu.VMEM((NCH, EC), jnp.int32),
            pltpu.VMEM((NSLOT * EC, D_H), jnp.float32),
            pltpu.VMEM((ZROWS, D_H), jnp.float32),
            pltpu.VMEM_SHARED((CHUNK_PAD, D_H), jnp.float32),
            pltpu.SemaphoreType.DMA,
            pltpu.SemaphoreType.DMA,
            pltpu.SemaphoreType.DMA,
        ],
        compiler_params=pltpu.CompilerParams(
            use_tc_tiling_on_sc=False, needs_layout_passes=False),
    )
    def seg_sum(m_hbm, ed_hbm, out_hbm, sbuf, dbuf, csrc,
                cdst, rows, zbuf, acc, sem_g, sem_s, sem_e):
        c = lax.axis_index("c")
        s = lax.axis_index("s")
        zero16 = jnp.zeros((16,), jnp.float32)
        for i in range(ZROWS):
            for col in range(D_H // 16):
                zbuf[i, pl.ds(col * 16, 16)] = zero16

        def slotref(q):
            sl = lax.rem(q, NSLOT)
            return rows.at[pl.ds(pl.multiple_of(sl * EC, EC), EC)]

        def fire_gather(q):
            pltpu.async_copy(m_hbm.at[csrc.at[q & (NCH - 1)]], slotref(q),
                             sem_g)

        def drain_gather(q):
            pltpu.make_async_copy(m_hbm.at[csrc.at[q & (NCH - 1)]],
                                  slotref(q), sem_g).wait()

        def fire_scatter(q):
            pltpu.async_copy(slotref(q), acc.at[cdst.at[q & (NCH - 1)]],
                             sem_s, add=True)

        def drain_scatter(q):
            pltpu.make_async_copy(slotref(q),
                                  acc.at[cdst.at[q & (NCH - 1)]],
                                  sem_s).wait()

        def pump(gq):
            @pl.when(gq >= 2 * SLAG)
            def _():
                drain_scatter(gq - 2 * SLAG)

            fire_gather(gq)

            @pl.when(gq >= SLAG)
            def _():
                drain_gather(gq - SLAG)
                fire_scatter(gq - SLAG)

        def chunk_body(k, _):
            base = (2 * c + k) * CHUNK
            trash = CHUNK + s

            for j in range(ROWS_PER_TILE // ZROWS):
                pltpu.sync_copy(
                    zbuf, acc.at[pl.ds(s * ROWS_PER_TILE + j * ZROWS, ZROWS)])

            def fire_edge_load(b):
                row_off = s * STRIPE_ROWS + b * BLK_ROWS
                buf = pl.ds((b & 1) * BLK_ROWS, BLK_ROWS)
                pltpu.async_copy(ed_hbm.at[pl.ds(row_off, BLK_ROWS)],
                                 sbuf.at[buf], sem_e)
                pltpu.async_copy(
                    ed_hbm.at[pl.ds(E2D_ROWS + row_off, BLK_ROWS)],
                    dbuf.at[buf], sem_e)

            def drain_edge_load(b):
                row_off = s * STRIPE_ROWS + b * BLK_ROWS
                buf = pl.ds((b & 1) * BLK_ROWS, BLK_ROWS)
                pltpu.make_async_copy(ed_hbm.at[pl.ds(row_off, BLK_ROWS)],
                                      sbuf.at[buf], sem_e).wait()
                pltpu.make_async_copy(
                    ed_hbm.at[pl.ds(E2D_ROWS + row_off, BLK_ROWS)],
                    dbuf.at[buf], sem_e).wait()

            fire_edge_load(0)
            plsc.subcore_barrier()

            def blk(b, carry):
                off0, gq0 = carry

                @pl.when(b + 1 < NBLK)
                def _():
                    fire_edge_load(b + 1)

                drain_edge_load(b)
                rbase = (b & 1) * BLK_ROWS

                def group(g, carry):
                    off, gq = carry
                    r = rbase + (g >> 3)
                    col = pl.multiple_of((g & 7) * 16, 16)
                    s16 = sbuf[r, pl.ds(col, 16)]
                    d16 = dbuf[r, pl.ds(col, 16)]
                    ok = (d16 >= base) & (d16 < base + CHUNK)
                    okc = ok.astype(jnp.int32)
                    inc = jnp.cumsum(okc)
                    pos = (off + inc - 1) & (CAP - 1)
                    prow = pos >> 7
                    pcol = pos & (EC - 1)
                    g16 = jnp.minimum(s16, N_NODES - 1)
                    plsc.store_scatter(csrc, [prow, pcol], g16, mask=ok)
                    plsc.store_scatter(cdst, [prow, pcol], d16 - base,
                                       mask=ok)
                    off = off + jnp.sum(okc, axis=0)
                    fire = (off >> 7) > gq

                    @pl.when(fire)
                    def _():
                        pump(gq)

                    return (off, jnp.where(fire, gq + 1, gq))

                return lax.fori_loop(0, BLK_ROWS * EC // 16, group,
                                     (off0, gq0))

            off, gq = lax.fori_loop(
                0, NBLK, blk, (jnp.int32(0), jnp.int32(0)))

            target = ((off + EC - 1) // EC) * EC

            def padg(i, _):
                pos_l = off + i * 16 + lax.iota(jnp.int32, 16)
                mk = pos_l < target
                posm = pos_l & (CAP - 1)
                prow = posm >> 7
                pcol = posm & (EC - 1)
                zi = jnp.zeros((16,), jnp.int32)
                plsc.store_scatter(csrc, [prow, pcol], zi, mask=mk)
                plsc.store_scatter(cdst, [prow, pcol], zi + trash, mask=mk)
                return 0

            lax.fori_loop(0, EC // 16, padg, 0)
            nchunks = target // EC

            def fire_rest(_, gq2):
                pump(gq2)
                return gq2 + 1

            gq = lax.fori_loop(0, nchunks - gq, fire_rest, gq)

            def ep(i, g2):
                @pl.when((g2 >= 2 * SLAG) & (g2 - 2 * SLAG < nchunks))
                def _():
                    drain_scatter(g2 - 2 * SLAG)

                @pl.when((g2 >= SLAG) & (g2 - SLAG < nchunks))
                def _():
                    drain_gather(g2 - SLAG)
                    fire_scatter(g2 - SLAG)

                return g2 + 1

            lax.fori_loop(0, 2 * SLAG, ep, gq)
            plsc.subcore_barrier()

            out_base = (2 * c + k) * CHUNK + s * ROWS_PER_TILE

            @pl.when(s < NUM_TILES - 1)
            def _():
                pltpu.sync_copy(
                    acc.at[pl.ds(s * ROWS_PER_TILE, ROWS_PER_TILE)],
                    out_hbm.at[pl.ds(out_base, ROWS_PER_TILE)])

            @pl.when(s == NUM_TILES - 1)
            def _():
                pltpu.sync_copy(
                    acc.at[pl.ds(s * ROWS_PER_TILE, LAST_TILE_ROWS)],
                    out_hbm.at[pl.ds(out_base, LAST_TILE_ROWS)])

            plsc.subcore_barrier()
            return 0

        lax.fori_loop(0, NCHUNKS // NUM_CORES, chunk_body, 0)

    return seg_sum


_seg_sum = _make_segment_sum()


NP = N_NODES // 2
RBP = 200


def _blkdiag(wt):
    d0, d1 = wt.shape
    z = jnp.zeros((d0, d1), jnp.float32)
    return jnp.concatenate(
        [jnp.concatenate([wt, z], axis=1), jnp.concatenate([z, wt], axis=1)],
        axis=0)


def _fc_pre(xp, wp_bd, b_bd, wrel_bd):
    def body(x_ref, wp_ref, b_ref, wr_ref, h_ref, m_ref):
        h = jnp.dot(x_ref[...], wp_ref[...],
                    preferred_element_type=jnp.float32) + b_ref[...]
        h_ref[...] = h
        m_ref[...] = jnp.dot(h, wr_ref[...], preferred_element_type=jnp.float32)

    return pl.pallas_call(
        body,
        grid=(NP // RBP,),
        in_specs=[
            pl.BlockSpec((RBP, 2 * D_IN), lambda i: (i, 0)),
            pl.BlockSpec((2 * D_IN, 2 * D_H), lambda i: (0, 0)),
            pl.BlockSpec((1, 2 * D_H), lambda i: (0, 0)),
            pl.BlockSpec((2 * D_H, 2 * D_H), lambda i: (0, 0)),
        ],
        out_specs=[
            pl.BlockSpec((RBP, 2 * D_H), lambda i: (i, 0)),
            pl.BlockSpec((RBP, 2 * D_H), lambda i: (i, 0)),
        ],
        out_shape=[
            jax.ShapeDtypeStruct((NP, 2 * D_H), jnp.float32),
            jax.ShapeDtypeStruct((NP, 2 * D_H), jnp.float32),
        ],
    )(xp, wp_bd, b_bd, wrel_bd)


def _gc_mid(aggp, hp_prev, wroot_bd, b_bd, wnrel_bd):
    def body(a_ref, h_ref, wr_ref, b_ref, wn_ref, o_ref, m_ref):
        t = jnp.tanh(a_ref[...] + b_ref[...] +
                     jnp.dot(h_ref[...], wr_ref[...],
                             preferred_element_type=jnp.float32))
        o_ref[...] = t
        m_ref[...] = jnp.dot(t, wn_ref[...], preferred_element_type=jnp.float32)

    return pl.pallas_call(
        body,
        grid=(NP // RBP,),
        in_specs=[
            pl.BlockSpec((RBP, 2 * D_H), lambda i: (i, 0)),
            pl.BlockSpec((RBP, 2 * D_H), lambda i: (i, 0)),
            pl.BlockSpec((2 * D_H, 2 * D_H), lambda i: (0, 0)),
            pl.BlockSpec((1, 2 * D_H), lambda i: (0, 0)),
            pl.BlockSpec((2 * D_H, 2 * D_H), lambda i: (0, 0)),
        ],
        out_specs=[
            pl.BlockSpec((RBP, 2 * D_H), lambda i: (i, 0)),
            pl.BlockSpec((RBP, 2 * D_H), lambda i: (i, 0)),
        ],
        out_shape=[
            jax.ShapeDtypeStruct((NP, 2 * D_H), jnp.float32),
            jax.ShapeDtypeStruct((NP, 2 * D_H), jnp.float32),
        ],
    )(aggp, hp_prev, wroot_bd, b_bd, wnrel_bd)


def _gc_last(aggp, hp_prev, wroot_bd, b_bd):
    def body(a_ref, h_ref, wr_ref, b_ref, o_ref):
        o_ref[...] = jnp.tanh(a_ref[...] + b_ref[...] +
                              jnp.dot(h_ref[...], wr_ref[...],
                                      preferred_element_type=jnp.float32))

    return pl.pallas_call(
        body,
        grid=(NP // RBP,),
        in_specs=[
            pl.BlockSpec((RBP, 2 * D_H), lambda i: (i, 0)),
            pl.BlockSpec((RBP, 2 * D_H), lambda i: (i, 0)),
            pl.BlockSpec((2 * D_H, 2 * D_H), lambda i: (0, 0)),
            pl.BlockSpec((1, 2 * D_H), lambda i: (0, 0)),
        ],
        out_specs=pl.BlockSpec((RBP, 2 * D_H), lambda i: (i, 0)),
        out_shape=jax.ShapeDtypeStruct((NP, 2 * D_H), jnp.float32),
    )(aggp, hp_prev, wroot_bd, b_bd)


def kernel(x, edge_index, W_pre, b_pre, W1_rel, W1_root, b1, W2_rel, W2_root,
           b2):
    ed2d = jnp.pad(edge_index, ((0, 0), (0, E_PAD - N_EDGES)),
                   constant_values=N_NODES).reshape(2 * E2D_ROWS, EC)

    xp = x.reshape(NP, 2 * D_IN)
    b0d = jnp.concatenate([b_pre, b_pre]).reshape(1, 2 * D_H)
    b1d = jnp.concatenate([b1, b1]).reshape(1, 2 * D_H)
    b2d = jnp.concatenate([b2, b2]).reshape(1, 2 * D_H)

    h0p, m1p = _fc_pre(xp, _blkdiag(W_pre.T), b0d, _blkdiag(W1_rel.T))
    agg1 = _seg_sum(m1p.reshape(N_NODES, D_H), ed2d)
    h1p, m2p = _gc_mid(agg1.reshape(NP, 2 * D_H), h0p,
                       _blkdiag(W1_root.T), b1d, _blkdiag(W2_rel.T))
    agg2 = _seg_sum(m2p.reshape(N_NODES, D_H), ed2d)
    h2p = _gc_last(agg2.reshape(NP, 2 * D_H), h1p, _blkdiag(W2_root.T), b2d)
    return h2p.reshape(N_NODES, D_H)
